# Initial kernel scaffold; baseline (speedup 1.0000x reference)
import functools
import math

import jax
import jax.numpy as jnp
from jax import lax
from jax.experimental import pallas as pl
from jax.experimental.pallas import tpu as pltpu
from jax.experimental.pallas import tpu_sc as plsc

_NC = 2
_NS = 16
_NW = _NC * _NS
_K = 80
_ZR = 125


def _sc_spmm(X, rows, cols, vals):
    N, D = X.shape
    E = rows.shape[0]
    per_w = E // _NW
    n_chunks = per_w // _K
    rows_per_tile = N // _NS
    assert per_w * _NW == E and n_chunks * _K == per_w
    assert rows_per_tile * _NS == N and rows_per_tile % _ZR == 0 and D % 16 == 0

    mesh = plsc.VectorSubcoreMesh(core_axis_name="c", subcore_axis_name="s")

    @functools.partial(
        pl.kernel,
        out_type=jax.ShapeDtypeStruct((_NC, N, D), jnp.float32),
        mesh=mesh,
        scratch_types=[
            pltpu.VMEM((_K,), jnp.int32),
            pltpu.VMEM((_K,), jnp.int32),
            pltpu.VMEM((_K,), jnp.float32),
            pltpu.VMEM((_K, D), jnp.float32),
            pltpu.VMEM((_ZR, D), jnp.float32),
            pltpu.VMEM_SHARED((N, D), jnp.float32),
            pltpu.SemaphoreType.DMA,
        ],
    )
    def spmm(x_hbm, rows_hbm, cols_hbm, vals_hbm, out_hbm,
             col_v, row_v, val_v, buf, zbuf, acc, sem):
        cid = lax.axis_index("c")
        sid = lax.axis_index("s")
        wid = sid * _NC + cid

        zero = jnp.zeros((16,), jnp.float32)

        def zrow(r, carry):
            for j in range(D // 16):
                zbuf[r, pl.ds(j * 16, 16)] = zero
            return carry

        lax.fori_loop(0, _ZR, zrow, 0)
        my_base = sid * rows_per_tile
        for t in range(rows_per_tile // _ZR):
            pltpu.sync_copy(zbuf, acc.at[pl.ds(my_base + t * _ZR, _ZR)])
        plsc.subcore_barrier()

        def chunk(i, carry):
            base = pl.multiple_of(wid * per_w + i * _K, 8)
            pltpu.sync_copy(cols_hbm.at[pl.ds(base, _K)], col_v)
            pltpu.sync_copy(rows_hbm.at[pl.ds(base, _K)], row_v)
            pltpu.sync_copy(vals_hbm.at[pl.ds(base, _K)], val_v)
            pltpu.async_copy(x_hbm.at[col_v], buf, sem).wait()

            def scale(r, c2):
                v = val_v[r]
                for j in range(D // 16):
                    sl = pl.ds(j * 16, 16)
                    buf[r, sl] = buf[r, sl] * v
                return c2

            lax.fori_loop(0, _K, scale, 0)
            pltpu.sync_copy(buf, acc.at[row_v], add=True)
            return carry

        lax.fori_loop(0, n_chunks, chunk, 0)

        plsc.subcore_barrier()
        pltpu.sync_copy(acc.at[pl.ds(my_base, rows_per_tile)],
                        out_hbm.at[cid, pl.ds(my_base, rows_per_tile)])

    return spmm(X, rows, cols, vals)


def _tc_combine(p0, p1, h0, W):
    N, D = h0.shape
    BN = 400
    beta = math.log(0.5 / 4 + 1)

    def body(p0_ref, p1_ref, h0_ref, w_ref, o_ref):
        support = 0.9 * (p0_ref[...] + p1_ref[...]) + 0.1 * h0_ref[...]
        o_ref[...] = beta * jnp.dot(support, w_ref[...],
                                    preferred_element_type=jnp.float32) \
            + (1.0 - beta) * support

    return pl.pallas_call(
        body,
        grid=(N // BN,),
        in_specs=[
            pl.BlockSpec((BN, D), lambda i: (i, 0)),
            pl.BlockSpec((BN, D), lambda i: (i, 0)),
            pl.BlockSpec((BN, D), lambda i: (i, 0)),
            pl.BlockSpec((D, D), lambda i: (0, 0)),
        ],
        out_specs=pl.BlockSpec((BN, D), lambda i: (i, 0)),
        out_shape=jax.ShapeDtypeStruct((N, D), jnp.float32),
    )(p0, p1, h0, W)


def kernel(X, h0, layer, edge_index, edge_vals, W):
    del layer
    rows = edge_index[0]
    cols = edge_index[1]
    partials = _sc_spmm(X, rows, cols, edge_vals)
    return _tc_combine(partials[0], partials[1], h0, W)


# speedup vs baseline: 4.0428x; 4.0428x over previous
import functools
import math

import jax
import jax.numpy as jnp
from jax import lax
from jax.experimental import pallas as pl
from jax.experimental.pallas import tpu as pltpu
from jax.experimental.pallas import tpu_sc as plsc

_NC = 2
_NS = 16
_NW = _NC * _NS
_K = 80
_RB = 80


def _sc_spmm(X, rows, cols, vals):
    N, D = X.shape
    E = rows.shape[0]
    per_w = E // _NW
    n_chunks = per_w // _K
    n_rb = N // _RB
    assert per_w * _NW == E and n_chunks * _K == per_w
    assert n_rb * _RB == N and D % 16 == 0 and _K % 16 == 0

    mesh = plsc.VectorSubcoreMesh(core_axis_name="c", subcore_axis_name="s")

    @functools.partial(
        pl.kernel,
        out_type=jax.ShapeDtypeStruct((_NC, N, D), jnp.float32),
        mesh=mesh,
        scratch_types=[
            pltpu.VMEM((_K,), jnp.int32),
            pltpu.VMEM((_K,), jnp.int32),
            pltpu.VMEM((_K,), jnp.float32),
            pltpu.VMEM((_K, D), jnp.float32),
            pltpu.VMEM((_RB, D), jnp.float32),
            pltpu.VMEM_SHARED((N, D), jnp.float32),
            pltpu.SemaphoreType.DMA,
        ],
    )
    def spmm(x_hbm, rows_hbm, cols_hbm, vals_hbm, out_hbm,
             col_v, row_v, val_v, buf, zbuf, acc, sem):
        cid = lax.axis_index("c")
        sid = lax.axis_index("s")
        wid = sid * _NC + cid

        my_nb = (n_rb // _NS) + jnp.where(sid < (n_rb % _NS), 1, 0)

        zero = jnp.zeros((16,), jnp.float32)

        def zrow(r, carry):
            for j in range(D // 16):
                zbuf[r, pl.ds(j * 16, 16)] = zero
            return carry

        lax.fori_loop(0, _RB, zrow, 0)

        def zblk(i, carry):
            off = pl.multiple_of((sid + i * _NS) * _RB, 8)
            pltpu.sync_copy(zbuf, acc.at[pl.ds(off, _RB)])
            return carry

        lax.fori_loop(0, my_nb, zblk, 0)
        plsc.subcore_barrier()

        def chunk(i, carry):
            base = pl.multiple_of(wid * per_w + i * _K, 8)
            pltpu.sync_copy(cols_hbm.at[pl.ds(base, _K)], col_v)
            pltpu.sync_copy(rows_hbm.at[pl.ds(base, _K)], row_v)
            pltpu.sync_copy(vals_hbm.at[pl.ds(base, _K)], val_v)
            pltpu.async_copy(x_hbm.at[col_v], buf, sem).wait()

            def scale(g, c2):
                v16 = val_v[pl.ds(g * 16, 16)]
                for r2 in range(16):
                    r = g * 16 + r2
                    v = v16[r2]
                    for j in range(D // 16):
                        sl = pl.ds(j * 16, 16)
                        buf[r, sl] = buf[r, sl] * v
                return c2

            lax.fori_loop(0, _K // 16, scale, 0)
            pltpu.sync_copy(buf, acc.at[row_v], add=True)
            return carry

        lax.fori_loop(0, n_chunks, chunk, 0)

        plsc.subcore_barrier()

        def oblk(i, carry):
            off = pl.multiple_of((sid + i * _NS) * _RB, 8)
            pltpu.sync_copy(acc.at[pl.ds(off, _RB)],
                            out_hbm.at[cid, pl.ds(off, _RB)])
            return carry

        lax.fori_loop(0, my_nb, oblk, 0)

    return spmm(X, rows, cols, vals)


def _tc_combine(p0, p1, h0, W):
    N, D = h0.shape
    BN = 400
    beta = math.log(0.5 / 4 + 1)

    def body(p0_ref, p1_ref, h0_ref, w_ref, o_ref):
        support = 0.9 * (p0_ref[...] + p1_ref[...]) + 0.1 * h0_ref[...]
        o_ref[...] = beta * jnp.dot(support, w_ref[...],
                                    preferred_element_type=jnp.float32) \
            + (1.0 - beta) * support

    return pl.pallas_call(
        body,
        grid=(N // BN,),
        in_specs=[
            pl.BlockSpec((BN, D), lambda i: (i, 0)),
            pl.BlockSpec((BN, D), lambda i: (i, 0)),
            pl.BlockSpec((BN, D), lambda i: (i, 0)),
            pl.BlockSpec((D, D), lambda i: (0, 0)),
        ],
        out_specs=pl.BlockSpec((BN, D), lambda i: (i, 0)),
        out_shape=jax.ShapeDtypeStruct((N, D), jnp.float32),
    )(p0, p1, h0, W)


def kernel(X, h0, layer, edge_index, edge_vals, W):
    del layer
    rows = edge_index[0]
    cols = edge_index[1]
    partials = _sc_spmm(X, rows, cols, edge_vals)
    return _tc_combine(partials[0], partials[1], h0, W)


# speedup vs baseline: 8.7265x; 2.1585x over previous
import functools
import math

import jax
import jax.numpy as jnp
from jax import lax
from jax.experimental import pallas as pl
from jax.experimental.pallas import tpu as pltpu
from jax.experimental.pallas import tpu_sc as plsc

_NC = 2
_NS = 16
_NW = _NC * _NS
_K = 80
_RB = 80


def _sc_spmm(X, rows4, cols4, vals4):
    N, D = X.shape
    n_blk, b_ch = rows4.shape[1], rows4.shape[2]
    n_rb = N // _RB
    assert rows4.shape == (_NW, n_blk, b_ch, _K)
    assert n_rb * _RB == N and D % 16 == 0 and _K % 16 == 0 and b_ch >= 2

    mesh = plsc.VectorSubcoreMesh(core_axis_name="c", subcore_axis_name="s")

    @functools.partial(
        pl.kernel,
        out_type=jax.ShapeDtypeStruct((_NC, N, D), jnp.float32),
        mesh=mesh,
        scratch_types=[
            pltpu.VMEM((b_ch, _K), jnp.int32),
            pltpu.VMEM((b_ch, _K), jnp.int32),
            pltpu.VMEM((b_ch, _K), jnp.float32),
            pltpu.VMEM((_K, D), jnp.float32),
            pltpu.VMEM((_K, D), jnp.float32),
            pltpu.VMEM_SHARED((N, D), jnp.float32),
            pltpu.SemaphoreType.DMA,
            pltpu.SemaphoreType.DMA,
        ],
    )
    def spmm(x_hbm, rows_hbm, cols_hbm, vals_hbm, out_hbm,
             col_blk, row_blk, val_blk, buf0, buf1, acc, sem0, sem1):
        cid = lax.axis_index("c")
        sid = lax.axis_index("s")
        wid = sid * _NC + cid
        bufs = (buf0, buf1)
        sems = (sem0, sem1)

        my_nb = (n_rb // _NS) + jnp.where(sid < (n_rb % _NS), 1, 0)
        zero = jnp.zeros((16,), jnp.float32)

        def zrow(r, carry):
            for j in range(D // 16):
                buf0[r, pl.ds(j * 16, 16)] = zero
            return carry

        lax.fori_loop(0, _RB, zrow, 0)

        def zblk(i, carry):
            off = pl.multiple_of((sid + i * _NS) * _RB, 8)
            pltpu.sync_copy(buf0, acc.at[pl.ds(off, _RB)])
            return carry

        lax.fori_loop(0, my_nb, zblk, 0)
        plsc.subcore_barrier()

        def start_gather(i, b):
            pltpu.async_copy(x_hbm.at[col_blk.at[i]], bufs[b], sems[b])

        def wait_gather(i, b):
            pltpu.make_async_copy(x_hbm.at[col_blk.at[i]], bufs[b],
                                  sems[b]).wait()

        def process(i, b):
            wait_gather(i, b)
            buf = bufs[b]

            def scale(g, c2):
                v16 = val_blk[i, pl.ds(g * 16, 16)]
                for r2 in range(16):
                    v = v16[r2]
                    for j in range(D // 16):
                        sl = pl.ds(j * 16, 16)
                        buf[g * 16 + r2, sl] = buf[g * 16 + r2, sl] * v
                return c2

            lax.fori_loop(0, _K // 16, scale, 0)
            pltpu.sync_copy(buf, acc.at[row_blk.at[i]], add=True)

        def block(t, carry):
            pltpu.sync_copy(cols_hbm.at[wid, t], col_blk)
            pltpu.sync_copy(rows_hbm.at[wid, t], row_blk)
            pltpu.sync_copy(vals_hbm.at[wid, t], val_blk)
            start_gather(0, 0)
            start_gather(1, 1)

            def pair(p, c2):
                for b in range(2):
                    i = 2 * p + b
                    process(i, b)
                    nxt = i + 2

                    @pl.when(nxt < b_ch)
                    def _():
                        start_gather(nxt, b)
                return c2

            lax.fori_loop(0, b_ch // 2, pair, 0)
            if b_ch % 2:
                process(b_ch - 1, (b_ch - 1) % 2)
            return carry

        lax.fori_loop(0, n_blk, block, 0)

        plsc.subcore_barrier()

        def oblk(i, carry):
            off = pl.multiple_of((sid + i * _NS) * _RB, 8)
            pltpu.sync_copy(acc.at[pl.ds(off, _RB)],
                            out_hbm.at[cid, pl.ds(off, _RB)])
            return carry

        lax.fori_loop(0, my_nb, oblk, 0)

    return spmm(X, rows4, cols4, vals4)


def _tc_combine(p0, p1, h0, W):
    N, D = h0.shape
    BN = 400
    beta = math.log(0.5 / 4 + 1)

    def body(p0_ref, p1_ref, h0_ref, w_ref, o_ref):
        support = 0.9 * (p0_ref[...] + p1_ref[...]) + 0.1 * h0_ref[...]
        o_ref[...] = beta * jnp.dot(support, w_ref[...],
                                    preferred_element_type=jnp.float32) \
            + (1.0 - beta) * support

    return pl.pallas_call(
        body,
        grid=(N // BN,),
        in_specs=[
            pl.BlockSpec((BN, D), lambda i: (i, 0)),
            pl.BlockSpec((BN, D), lambda i: (i, 0)),
            pl.BlockSpec((BN, D), lambda i: (i, 0)),
            pl.BlockSpec((D, D), lambda i: (0, 0)),
        ],
        out_specs=pl.BlockSpec((BN, D), lambda i: (i, 0)),
        out_shape=jax.ShapeDtypeStruct((N, D), jnp.float32),
    )(p0, p1, h0, W)


def kernel(X, h0, layer, edge_index, edge_vals, W):
    del layer
    E = edge_vals.shape[0]
    n_blk = 5
    n_chunks = E // (_NW * _K)
    b_ch = n_chunks // n_blk
    assert b_ch * n_blk * _NW * _K == E
    shape4 = (_NW, n_blk, b_ch, _K)
    rows4 = edge_index[0].reshape(shape4)
    cols4 = edge_index[1].reshape(shape4)
    vals4 = edge_vals.reshape(shape4)
    partials = _sc_spmm(X, rows4, cols4, vals4)
    return _tc_combine(partials[0], partials[1], h0, W)
